# baseline (speedup 1.0000x reference)
import jax
import jax.numpy as jnp
from jax.experimental import pallas as pl
from jax.experimental.pallas import tpu as pltpu

_B = 8
_N = 2048
_D = 1024
_NMIX = 4
_CN = 1024


def _fused_kernel(eps_ref, logits_ref, mu_ref, sig_ref, w1_ref, b1_ref,
                  w2_ref, b2_ref, out_ref):
    logit = [logits_ref[0, k] for k in range(_NMIX)]
    m = logit[0]
    for k in range(1, _NMIX):
        m = jnp.maximum(m, logit[k])
    ex = [jnp.exp(l - m) for l in logit]
    s = ex[0]
    for k in range(1, _NMIX):
        s = s + ex[k]
    w = [e / s for e in ex]

    acc = eps_ref[0, :, 0, :] * (sig_ref[0:1, :] * w[0])
    cvec = mu_ref[0:1, :] * w[0]
    for k in range(1, _NMIX):
        acc += eps_ref[0, :, k, :] * (sig_ref[k:k + 1, :] * w[k])
        cvec += mu_ref[k:k + 1, :] * w[k]
    x = acc + cvec

    h = jax.lax.dot_general(x, w1_ref[...], (((1,), (1,)), ((), ())),
                            preferred_element_type=jnp.float32)
    h += b1_ref[...]
    h *= jax.nn.sigmoid(h)

    yt = jax.lax.dot_general(w2_ref[...], h, (((1,), (1,)), ((), ())),
                             preferred_element_type=jnp.float32)
    out_ref[0] = yt + b2_ref[...].reshape(_D, 1)


@jax.jit
def kernel(output_sizes, eps, logits, mu, sig, W1, b1, W2, b2):
    del output_sizes
    grid = (_B, _N // _CN)
    out = pl.pallas_call(
        _fused_kernel,
        grid=grid,
        in_specs=[
            pl.BlockSpec((1, _CN, _NMIX, _D), lambda b, j: (b, j, 0, 0)),
            pl.BlockSpec((1, _NMIX), lambda b, j: (0, 0)),
            pl.BlockSpec((_NMIX, _D), lambda b, j: (0, 0)),
            pl.BlockSpec((_NMIX, _D), lambda b, j: (0, 0)),
            pl.BlockSpec((_D, _D), lambda b, j: (0, 0)),
            pl.BlockSpec((1, _D), lambda b, j: (0, 0)),
            pl.BlockSpec((_D, _D), lambda b, j: (0, 0)),
            pl.BlockSpec((1, _D), lambda b, j: (0, 0)),
        ],
        out_specs=pl.BlockSpec((1, _D, _CN), lambda b, j: (b, 0, j)),
        out_shape=jax.ShapeDtypeStruct((_B, _D, _N), jnp.float32),
        compiler_params=pltpu.CompilerParams(
            dimension_semantics=("parallel", "parallel")),
    )(eps, logits.reshape(1, _NMIX), mu, sig,
      W1, b1.reshape(1, _D), W2, b2.reshape(1, _D))
    return out


# speedup vs baseline: 1.0005x; 1.0005x over previous
import jax
import jax.numpy as jnp
from jax.experimental import pallas as pl
from jax.experimental.pallas import tpu as pltpu

_B = 8
_N = 2048
_D = 1024
_NMIX = 4
_CN = 1024


def _fused_kernel(eps_ref, logits_ref, mu_ref, sig_ref, w1_ref, b1_ref,
                  w2_ref, b2_ref, out_ref):
    logit = [logits_ref[0, k] for k in range(_NMIX)]
    m = logit[0]
    for k in range(1, _NMIX):
        m = jnp.maximum(m, logit[k])
    ex = [jnp.exp(l - m) for l in logit]
    s = ex[0]
    for k in range(1, _NMIX):
        s = s + ex[k]
    w = [e / s for e in ex]

    acc = eps_ref[:, 0, :] * (sig_ref[0:1, :] * w[0])
    cvec = mu_ref[0:1, :] * w[0]
    for k in range(1, _NMIX):
        acc += eps_ref[:, k, :] * (sig_ref[k:k + 1, :] * w[k])
        cvec += mu_ref[k:k + 1, :] * w[k]
    x = acc + cvec

    h = jax.lax.dot_general(x, w1_ref[...], (((1,), (1,)), ((), ())),
                            preferred_element_type=jnp.float32)
    h += b1_ref[...]
    h *= jax.nn.sigmoid(h)

    yt = jax.lax.dot_general(w2_ref[...], h, (((1,), (1,)), ((), ())),
                             preferred_element_type=jnp.float32)
    out_ref[0] = yt + b2_ref[...].reshape(_D, 1)


@jax.jit
def kernel(output_sizes, eps, logits, mu, sig, W1, b1, W2, b2):
    del output_sizes
    grid = (_B * _N // _CN,)
    out = pl.pallas_call(
        _fused_kernel,
        grid=grid,
        in_specs=[
            pl.BlockSpec((_CN, _NMIX, _D), lambda t: (t, 0, 0)),
            pl.BlockSpec((1, _NMIX), lambda t: (0, 0)),
            pl.BlockSpec((_NMIX, _D), lambda t: (0, 0)),
            pl.BlockSpec((_NMIX, _D), lambda t: (0, 0)),
            pl.BlockSpec((_D, _D), lambda t: (0, 0)),
            pl.BlockSpec((1, _D), lambda t: (0, 0)),
            pl.BlockSpec((_D, _D), lambda t: (0, 0)),
            pl.BlockSpec((1, _D), lambda t: (0, 0)),
        ],
        out_specs=pl.BlockSpec((1, _D, _CN),
                               lambda t: (t // (_N // _CN), 0,
                                          t % (_N // _CN))),
        out_shape=jax.ShapeDtypeStruct((_B, _D, _N), jnp.float32),
        compiler_params=pltpu.CompilerParams(
            dimension_semantics=("parallel",)),
    )(eps.reshape(_B * _N, _NMIX, _D), logits.reshape(1, _NMIX), mu, sig,
      W1, b1.reshape(1, _D), W2, b2.reshape(1, _D))
    return out
